# baseline (speedup 1.0000x reference)
import jax
import jax.numpy as jnp
from jax.experimental import pallas as pl

_B, _N, _FD = 8, 4096, 128
_M, _K = 1024, 32
_GH, _FH = 256, 256

_ROWS = 512


def _fps(points, m):
    n = points.shape[0]

    def body(i, state):
        idx, dists = state
        last = points[idx[i - 1]]
        d = jnp.sum((points - last) ** 2, axis=1)
        dists = jnp.minimum(dists, d)
        idx = idx.at[i].set(jnp.argmax(dists).astype(jnp.int32))
        return idx, dists

    idx = jnp.zeros(m, dtype=jnp.int32)
    dists = jnp.full(n, jnp.float32(1e10))
    idx, _ = jax.lax.fori_loop(1, m, body, (idx, dists))
    return idx


def _mlp_block(xg_ref, xf_ref, w1gt_ref, w2gt_ref, w2ge_ref, w1ft_ref, b1f_ref,
               w2ft_ref, b2f_ref, out_ref):
    h = jnp.dot(xg_ref[...], w1gt_ref[...], preferred_element_type=jnp.float32)
    e2 = -0.5 * jnp.sum(h * h, axis=1, keepdims=True)
    xgeo = (jnp.dot(h, w2gt_ref[...], preferred_element_type=jnp.float32)
            - w2ge_ref[0, :][None, :] + e2 * w2ge_ref[1, :][None, :])
    hf = jnp.maximum(
        jnp.dot(xf_ref[...], w1ft_ref[...], preferred_element_type=jnp.float32)
        + b1f_ref[...], 0.0)
    xfeat = jnp.dot(hf, w2ft_ref[...], preferred_element_type=jnp.float32) + b2f_ref[...]
    out_ref[:, :_GH] = xgeo
    out_ref[:, _GH:] = xfeat


def _mlps(xg, xf, W1g, W2g, W1f, b1f, W2f, b2f):
    nrows = xg.shape[0]
    grid = nrows // _ROWS
    w2gt = W2g[:, :_GH].T
    w2ge = W2g[:, _GH:].T
    out = pl.pallas_call(
        _mlp_block,
        grid=(grid,),
        in_specs=[
            pl.BlockSpec((_ROWS, _K * 5), lambda i: (i, 0)),
            pl.BlockSpec((_ROWS, _K * _FD), lambda i: (i, 0)),
            pl.BlockSpec((_K * 5, _GH), lambda i: (0, 0)),
            pl.BlockSpec((_GH, _GH), lambda i: (0, 0)),
            pl.BlockSpec((2, _GH), lambda i: (0, 0)),
            pl.BlockSpec((_K * _FD, _FH), lambda i: (0, 0)),
            pl.BlockSpec((1, _FH), lambda i: (0, 0)),
            pl.BlockSpec((_FH, _FH), lambda i: (0, 0)),
            pl.BlockSpec((1, _FH), lambda i: (0, 0)),
        ],
        out_specs=pl.BlockSpec((_ROWS, _GH + _FH), lambda i: (i, 0)),
        out_shape=jax.ShapeDtypeStruct((nrows, _GH + _FH), jnp.float32),
    )(xg, xf, W1g.T, w2gt, w2ge, W1f.T, b1f.reshape(1, _FH), W2f.T,
      b2f.reshape(1, _FH))
    return out


def kernel(xyz, features, W1g, W2g, W1f, b1f, W2f, b2f):
    b, n, _ = xyz.shape
    idx = jax.vmap(lambda p: _fps(p, _M))(xyz)
    centroids = jnp.take_along_axis(xyz, idx[:, :, None], axis=1)
    d2 = (jnp.sum(centroids ** 2, axis=-1)[:, :, None]
          + jnp.sum(xyz ** 2, axis=-1)[:, None, :]
          - 2.0 * jnp.einsum('bmd,bnd->bmn', centroids, xyz))
    _, group_idx = jax.lax.top_k(-d2, _K)
    grouped_xyz = jax.vmap(lambda p, gi: p[gi])(xyz, group_idx)
    grouped_xyz = grouped_xyz - centroids[:, :, None, :]
    x = grouped_xyz.reshape(b * _M, _K, 3)
    e1 = -jnp.ones((b * _M, _K, 1), dtype=x.dtype)
    e2 = -jnp.sum(x ** 2, axis=2, keepdims=True) / 2.0
    xg = jnp.concatenate([x, e1, e2], axis=2).reshape(b * _M, _K * 5)
    grouped_feat = jax.vmap(lambda f, gi: f[gi])(features, group_idx)
    xf = grouped_feat.reshape(b * _M, _K * _FD)
    out = _mlps(xg, xf, W1g, W2g, W1f, b1f, W2f, b2f)
    return out.reshape(b, _M, _GH + _FH)


# speedup vs baseline: 1.6443x; 1.6443x over previous
import jax
import jax.numpy as jnp
from jax.experimental import pallas as pl

_B, _N, _FD = 8, 4096, 128
_M, _K = 1024, 32
_GH, _FH = 256, 256

_ROWS = 512


def _fps_body(x_ref, y_ref, z_ref, cx_ref, cy_ref, cz_ref):
    X = x_ref[...]
    Y = y_ref[...]
    Z = z_ref[...]
    lane = jax.lax.broadcasted_iota(jnp.int32, (_B, _N), 1)
    big = jnp.int32(_N)

    mlane = jax.lax.broadcasted_iota(jnp.int32, (_B, _M), 1)

    first = lane == 0
    cx0 = jnp.sum(jnp.where(first, X, 0.0), axis=1, keepdims=True)
    cy0 = jnp.sum(jnp.where(first, Y, 0.0), axis=1, keepdims=True)
    cz0 = jnp.sum(jnp.where(first, Z, 0.0), axis=1, keepdims=True)
    zerosm = jnp.zeros((_B, _M), jnp.float32)
    cxs0 = jnp.where(mlane == 0, cx0, zerosm)
    cys0 = jnp.where(mlane == 0, cy0, zerosm)
    czs0 = jnp.where(mlane == 0, cz0, zerosm)

    dists0 = jnp.full((_B, _N), 1e10, dtype=jnp.float32)

    def body(i, state):
        dists, lx, ly, lz, cxs, cys, czs = state
        d = (X - lx) ** 2 + (Y - ly) ** 2 + (Z - lz) ** 2
        dists = jnp.minimum(dists, d)
        m = jnp.max(dists, axis=1, keepdims=True)
        jstar = jnp.min(jnp.where(dists == m, lane, big), axis=1, keepdims=True)
        sel = lane == jstar
        nx = jnp.sum(jnp.where(sel, X, 0.0), axis=1, keepdims=True)
        ny = jnp.sum(jnp.where(sel, Y, 0.0), axis=1, keepdims=True)
        nz = jnp.sum(jnp.where(sel, Z, 0.0), axis=1, keepdims=True)
        here = mlane == i
        cxs = jnp.where(here, nx, cxs)
        cys = jnp.where(here, ny, cys)
        czs = jnp.where(here, nz, czs)
        return dists, nx, ny, nz, cxs, cys, czs

    _, _, _, _, cxs, cys, czs = jax.lax.fori_loop(
        1, _M, body, (dists0, cx0, cy0, cz0, cxs0, cys0, czs0))
    cx_ref[...] = cxs
    cy_ref[...] = cys
    cz_ref[...] = czs


def _fps_centroids(xyz):
    x = xyz[:, :, 0]
    y = xyz[:, :, 1]
    z = xyz[:, :, 2]
    shp = jax.ShapeDtypeStruct((_B, _M), jnp.float32)
    cx, cy, cz = pl.pallas_call(
        _fps_body,
        out_shape=(shp, shp, shp),
    )(x, y, z)
    return jnp.stack([cx, cy, cz], axis=-1)


def _mlp_block(xg_ref, xf_ref, w1gt_ref, w2gt_ref, w2ge_ref, w1ft_ref, b1f_ref,
               w2ft_ref, b2f_ref, out_ref):
    h = jnp.dot(xg_ref[...], w1gt_ref[...], preferred_element_type=jnp.float32)
    e2 = -0.5 * jnp.sum(h * h, axis=1, keepdims=True)
    xgeo = (jnp.dot(h, w2gt_ref[...], preferred_element_type=jnp.float32)
            - w2ge_ref[0, :][None, :] + e2 * w2ge_ref[1, :][None, :])
    hf = jnp.maximum(
        jnp.dot(xf_ref[...], w1ft_ref[...], preferred_element_type=jnp.float32)
        + b1f_ref[...], 0.0)
    xfeat = jnp.dot(hf, w2ft_ref[...], preferred_element_type=jnp.float32) + b2f_ref[...]
    out_ref[:, :_GH] = xgeo
    out_ref[:, _GH:] = xfeat


def _mlps(xg, xf, W1g, W2g, W1f, b1f, W2f, b2f):
    nrows = xg.shape[0]
    grid = nrows // _ROWS
    w2gt = W2g[:, :_GH].T
    w2ge = W2g[:, _GH:].T
    out = pl.pallas_call(
        _mlp_block,
        grid=(grid,),
        in_specs=[
            pl.BlockSpec((_ROWS, _K * 5), lambda i: (i, 0)),
            pl.BlockSpec((_ROWS, _K * _FD), lambda i: (i, 0)),
            pl.BlockSpec((_K * 5, _GH), lambda i: (0, 0)),
            pl.BlockSpec((_GH, _GH), lambda i: (0, 0)),
            pl.BlockSpec((2, _GH), lambda i: (0, 0)),
            pl.BlockSpec((_K * _FD, _FH), lambda i: (0, 0)),
            pl.BlockSpec((1, _FH), lambda i: (0, 0)),
            pl.BlockSpec((_FH, _FH), lambda i: (0, 0)),
            pl.BlockSpec((1, _FH), lambda i: (0, 0)),
        ],
        out_specs=pl.BlockSpec((_ROWS, _GH + _FH), lambda i: (i, 0)),
        out_shape=jax.ShapeDtypeStruct((nrows, _GH + _FH), jnp.float32),
    )(xg, xf, W1g.T, w2gt, w2ge, W1f.T, b1f.reshape(1, _FH), W2f.T,
      b2f.reshape(1, _FH))
    return out


def kernel(xyz, features, W1g, W2g, W1f, b1f, W2f, b2f):
    b, n, _ = xyz.shape
    centroids = _fps_centroids(xyz)
    d2 = (jnp.sum(centroids ** 2, axis=-1)[:, :, None]
          + jnp.sum(xyz ** 2, axis=-1)[:, None, :]
          - 2.0 * jnp.einsum('bmd,bnd->bmn', centroids, xyz))
    _, group_idx = jax.lax.top_k(-d2, _K)
    grouped_xyz = jax.vmap(lambda p, gi: p[gi])(xyz, group_idx)
    grouped_xyz = grouped_xyz - centroids[:, :, None, :]
    x = grouped_xyz.reshape(b * _M, _K, 3)
    e1 = -jnp.ones((b * _M, _K, 1), dtype=x.dtype)
    e2 = -jnp.sum(x ** 2, axis=2, keepdims=True) / 2.0
    xg = jnp.concatenate([x, e1, e2], axis=2).reshape(b * _M, _K * 5)
    grouped_feat = jax.vmap(lambda f, gi: f[gi])(features, group_idx)
    xf = grouped_feat.reshape(b * _M, _K * _FD)
    out = _mlps(xg, xf, W1g, W2g, W1f, b1f, W2f, b2f)
    return out.reshape(b, _M, _GH + _FH)
